# baseline (speedup 1.0000x reference)
import dataclasses

import jax
import jax.numpy as jnp
from jax import lax
from jax.experimental import pallas as pl
from jax.experimental.pallas import tpu as pltpu
from jax.experimental.pallas import tpu_sc as plsc

B = 512
D = 128

N_ATOMS = 10000
N_BONDS = 160000

BBLK = 128
ABLK = 80
NB_BOND = N_BONDS // BBLK
NB_ATOM = N_ATOMS // ABLK
NW = 32
BOND_FLOOR = NB_BOND // NW
ATOM_FLOOR = NB_ATOM // NW
BOND_MAX = BOND_FLOOR + 1
ATOM_MAX = ATOM_FLOOR + 1


def _sc_pool_body(
    bond_hbm,
    bseg_hbm,
    atom_hbm,
    aseg_hbm,
    out_a_hbm,
    out_b_hbm,
    acc_a,
    acc_b,
    rows0,
    rows1,
    rows2,
    arows0,
    arows1,
    idxA,
    idxB,
    idxC,
    bseg_all,
    aseg_all,
    tmp_v,
    sem_pre,
    gsem0,
    gsem1,
    gsem2,
    ssem0,
    ssem1,
    ssem2,
    asem0,
    asem1,
):
    cid = lax.axis_index("c")
    sid = lax.axis_index("s")
    wid = sid * 2 + cid

    nb = BOND_FLOOR + jnp.where(wid < NB_BOND - BOND_FLOOR * NW, 1, 0)
    na = ATOM_FLOOR + jnp.where(wid < NB_ATOM - ATOM_FLOOR * NW, 1, 0)

    @pl.loop(0, nb)
    def _(j):
        row0 = (j * NW + wid) * BBLK
        pltpu.async_copy(bseg_hbm.at[pl.ds(row0, BBLK)], bseg_all.at[j], sem_pre)

    @pl.loop(0, na)
    def _(j):
        row0 = (j * NW + wid) * ABLK
        pltpu.async_copy(aseg_hbm.at[pl.ds(row0, ABLK)], aseg_all.at[j], sem_pre)

    @pl.loop(0, 32)
    def _(r):
        @pl.loop(0, D // 16)
        def _(c):
            tmp_v[r, pl.ds(c * 16, 16)] = jnp.zeros((16,), jnp.float32)

    pltpu.sync_copy(tmp_v, acc_a.at[pl.ds(sid * 32, 32)])
    pltpu.sync_copy(tmp_v, acc_b.at[pl.ds(sid * 32, 32)])

    @pl.loop(0, nb)
    def _(j):
        pltpu.make_async_copy(bseg_hbm.at[pl.ds(0, BBLK)], bseg_all.at[0], sem_pre).wait()

    lane = lax.broadcasted_iota(jnp.int32, (16,), 0)
    for c in range(BBLK // 16):
        idxA[pl.ds(c * 16, 16)] = 2 * (wid * BBLK + c * 16) + 2 * lane
        idxB[pl.ds(c * 16, 16)] = 2 * ((NW + wid) * BBLK + c * 16) + 2 * lane
        idxC[pl.ds(c * 16, 16)] = 2 * ((2 * NW + wid) * BBLK + c * 16) + 2 * lane

    @pl.loop(0, na)
    def _(j):
        pltpu.make_async_copy(aseg_hbm.at[pl.ds(0, ABLK)], aseg_all.at[0], sem_pre).wait()

    plsc.subcore_barrier()

    pltpu.async_copy(atom_hbm.at[pl.ds(wid * ABLK, ABLK)], arows0, asem0)
    pltpu.async_copy(atom_hbm.at[pl.ds((NW + wid) * ABLK, ABLK)], arows1, asem1)

    rowsv = [rows0, rows1, rows2]
    idxv = [idxA, idxB, idxC]
    gsemv = [gsem0, gsem1, gsem2]
    ssemv = [ssem0, ssem1, ssem2]

    def bond_step(j, o):
        jj = j + o
        k = o % 3
        k2 = (o + 2) % 3

        @pl.when(jj < nb)
        def _():
            pltpu.make_async_copy(bond_hbm.at[idxv[k]], rowsv[k], gsemv[k]).wait()
            for c in range(BBLK // 16):
                idxv[k][pl.ds(c * 16, 16)] = (
                    idxv[k][pl.ds(c * 16, 16)] + 6 * NW * BBLK
                )

            @pl.when(jj + 2 < nb)
            def _():
                @pl.when(jj >= 1)
                def _():
                    pltpu.make_async_copy(
                        rowsv[k2], acc_b.at[bseg_all.at[0]], ssemv[k2]
                    ).wait()

                pltpu.async_copy(bond_hbm.at[idxv[k2]], rowsv[k2], gsemv[k2])

            pltpu.async_copy(rowsv[k], acc_b.at[bseg_all.at[jj]], ssemv[k], add=True)

    pltpu.async_copy(bond_hbm.at[idxA], rows0, gsem0)
    pltpu.async_copy(bond_hbm.at[idxB], rows1, gsem1)

    @pl.loop(0, BOND_MAX + 2, step=3)
    def _(j):
        bond_step(j, 0)
        bond_step(j, 1)
        bond_step(j, 2)

    for k in range(3):
        pltpu.make_async_copy(rowsv[k], acc_b.at[bseg_all.at[0]], ssemv[k]).wait()

    arowsv = [arows0, arows1]
    asemv = [asem0, asem1]

    def atom_step(j, k):
        @pl.when(j < na)
        def _():
            pltpu.make_async_copy(
                atom_hbm.at[pl.ds(0, ABLK)], arowsv[k], asemv[k]
            ).wait()
            pltpu.sync_copy(arowsv[k], acc_a.at[aseg_all.at[j]], add=True)

            @pl.when(j + 2 < na)
            def _():
                row0 = ((j + 2) * NW + wid) * ABLK
                pltpu.async_copy(atom_hbm.at[pl.ds(row0, ABLK)], arowsv[k], asemv[k])

    @pl.loop(0, ATOM_MAX, step=2)
    def _(j):
        atom_step(j, 0)
        atom_step(j + 1, 1)

    plsc.subcore_barrier()

    pltpu.sync_copy(acc_a.at[pl.ds(sid * 32, 32)], tmp_v)
    pltpu.sync_copy(tmp_v, out_a_hbm.at[cid, pl.ds(sid * 32, 32)])
    pltpu.sync_copy(acc_b.at[pl.ds(sid * 32, 32)], tmp_v)
    pltpu.sync_copy(tmp_v, out_b_hbm.at[cid, pl.ds(sid * 32, 32)])


def _sc_pool(bond_feats, b_ids, atom_feats, a_ids):
    mesh = plsc.VectorSubcoreMesh(core_axis_name="c", subcore_axis_name="s")
    f32 = jnp.float32
    i32 = jnp.int32
    cp = pltpu.CompilerParams()
    if "needs_layout_passes" in pltpu.CompilerParams.__dataclass_fields__:
        cp = dataclasses.replace(cp, needs_layout_passes=False)
    kern = pl.kernel(
        _sc_pool_body,
        compiler_params=cp,
        out_type=(
            jax.ShapeDtypeStruct((2, B, D), f32),
            jax.ShapeDtypeStruct((2, B, D), f32),
        ),
        mesh=mesh,
        scratch_types=[
            pltpu.VMEM_SHARED((B, D), f32),
            pltpu.VMEM_SHARED((B, D), f32),
            pltpu.VMEM((BBLK, D), f32),
            pltpu.VMEM((BBLK, D), f32),
            pltpu.VMEM((BBLK, D), f32),
            pltpu.VMEM((ABLK, D), f32),
            pltpu.VMEM((ABLK, D), f32),
            pltpu.VMEM((BBLK,), i32),
            pltpu.VMEM((BBLK,), i32),
            pltpu.VMEM((BBLK,), i32),
            pltpu.VMEM((BOND_MAX, BBLK), i32),
            pltpu.VMEM((ATOM_MAX, ABLK), i32),
            pltpu.VMEM((32, D), f32),
            pltpu.SemaphoreType.DMA,
            pltpu.SemaphoreType.DMA,
            pltpu.SemaphoreType.DMA,
            pltpu.SemaphoreType.DMA,
            pltpu.SemaphoreType.DMA,
            pltpu.SemaphoreType.DMA,
            pltpu.SemaphoreType.DMA,
            pltpu.SemaphoreType.DMA,
            pltpu.SemaphoreType.DMA,
        ],
    )
    return kern(bond_feats, b_ids, atom_feats, a_ids)


def _combine_body(pa_ref, pb_ref, g_ref, out_ref):
    out_ref[:, 0:D] = pa_ref[0] + pa_ref[1]
    out_ref[:, D : 2 * D] = pb_ref[0] + pb_ref[1]
    out_ref[:, 2 * D : 3 * D] = g_ref[...]


def _combine(pa, pb, g):
    return pl.pallas_call(
        _combine_body,
        out_shape=jax.ShapeDtypeStruct((B, 3 * D), jnp.float32),
    )(pa, pb, g)


def kernel(atom_feats, bond_feats, global_feats, atom_segment_ids, bond_segment_ids):
    a_ids = atom_segment_ids.astype(jnp.int32)
    b_ids = bond_segment_ids.astype(jnp.int32)
    pa, pb = _sc_pool(bond_feats, b_ids, atom_feats, a_ids)
    return _combine(pa, pb, global_feats)


# speedup vs baseline: 1.0079x; 1.0079x over previous
import dataclasses

import jax
import jax.numpy as jnp
from jax import lax
from jax.experimental import pallas as pl
from jax.experimental.pallas import tpu as pltpu
from jax.experimental.pallas import tpu_sc as plsc

B = 512
D = 128

N_ATOMS = 10000
N_BONDS = 160000

BBLK = 128
ABLK = 80
NB_BOND = N_BONDS // BBLK
NB_ATOM = N_ATOMS // ABLK
NW = 32
BOND_FLOOR = NB_BOND // NW
ATOM_FLOOR = NB_ATOM // NW
BOND_MAX = BOND_FLOOR + 1
ATOM_MAX = ATOM_FLOOR + 1


def _sc_pool_body(
    bond_hbm,
    bseg_hbm,
    atom_hbm,
    aseg_hbm,
    out_a_hbm,
    out_b_hbm,
    acc_a,
    acc_b,
    rows0,
    rows1,
    rows2,
    arows0,
    arows1,
    idxA,
    idxB,
    idxC,
    bseg_all,
    aseg_all,
    tmp_v,
    sem_pre,
    gsem0,
    gsem1,
    gsem2,
    ssem0,
    ssem1,
    ssem2,
    asem0,
    asem1,
):
    cid = lax.axis_index("c")
    sid = lax.axis_index("s")
    wid = sid * 2 + cid

    nb = BOND_FLOOR + jnp.where(wid < NB_BOND - BOND_FLOOR * NW, 1, 0)
    na = ATOM_FLOOR + jnp.where(wid < NB_ATOM - ATOM_FLOOR * NW, 1, 0)

    @pl.loop(0, nb)
    def _(j):
        row0 = (j * NW + wid) * BBLK
        pltpu.async_copy(bseg_hbm.at[pl.ds(row0, BBLK)], bseg_all.at[j], sem_pre)

    @pl.loop(0, na)
    def _(j):
        row0 = (j * NW + wid) * ABLK
        pltpu.async_copy(aseg_hbm.at[pl.ds(row0, ABLK)], aseg_all.at[j], sem_pre)

    @pl.loop(0, 32)
    def _(r):
        @pl.loop(0, D // 16)
        def _(c):
            tmp_v[r, pl.ds(c * 16, 16)] = jnp.zeros((16,), jnp.float32)

    pltpu.sync_copy(tmp_v, acc_a.at[pl.ds(sid * 32, 32)])
    pltpu.sync_copy(tmp_v, acc_b.at[pl.ds(sid * 32, 32)])

    @pl.loop(0, nb)
    def _(j):
        pltpu.make_async_copy(bseg_hbm.at[pl.ds(0, BBLK)], bseg_all.at[0], sem_pre).wait()

    lane = lax.broadcasted_iota(jnp.int32, (16,), 0)
    for c in range(BBLK // 16):
        idxA[pl.ds(c * 16, 16)] = 2 * (wid * BBLK + c * 16) + 2 * lane
        idxB[pl.ds(c * 16, 16)] = 2 * ((NW + wid) * BBLK + c * 16) + 2 * lane
        idxC[pl.ds(c * 16, 16)] = 2 * ((2 * NW + wid) * BBLK + c * 16) + 2 * lane

    @pl.loop(0, na)
    def _(j):
        pltpu.make_async_copy(aseg_hbm.at[pl.ds(0, ABLK)], aseg_all.at[0], sem_pre).wait()

    plsc.subcore_barrier()

    rowsv = [rows0, rows1, rows2]
    idxv = [idxA, idxB, idxC]
    gsemv = [gsem0, gsem1, gsem2]
    ssemv = [ssem0, ssem1, ssem2]

    def bond_step(j, o):
        jj = j + o
        k = o % 3
        k2 = (o + 2) % 3

        @pl.when(jj < nb)
        def _():
            pltpu.make_async_copy(bond_hbm.at[idxv[k]], rowsv[k], gsemv[k]).wait()
            for c in range(BBLK // 16):
                idxv[k][pl.ds(c * 16, 16)] = (
                    idxv[k][pl.ds(c * 16, 16)] + 6 * NW * BBLK
                )

            @pl.when(jj + 2 < nb)
            def _():
                @pl.when(jj >= 1)
                def _():
                    pltpu.make_async_copy(
                        rowsv[k2], acc_b.at[bseg_all.at[0]], ssemv[k2]
                    ).wait()

                pltpu.async_copy(bond_hbm.at[idxv[k2]], rowsv[k2], gsemv[k2])

            pltpu.async_copy(rowsv[k], acc_b.at[bseg_all.at[jj]], ssemv[k], add=True)

    pltpu.async_copy(bond_hbm.at[idxA], rows0, gsem0)
    pltpu.async_copy(bond_hbm.at[idxB], rows1, gsem1)

    @pl.loop(0, BOND_MAX + 2, step=3)
    def _(j):
        bond_step(j, 0)
        bond_step(j, 1)
        bond_step(j, 2)

    for k in range(3):
        pltpu.make_async_copy(rowsv[k], acc_b.at[bseg_all.at[0]], ssemv[k]).wait()

    def atom_gather(j, buf, sem):
        row0 = (j * NW + wid) * ABLK
        pltpu.async_copy(atom_hbm.at[pl.ds(row0, ABLK)], buf, sem)

    def atom_step(j, buf, sem, nxt_buf, nxt_sem):
        @pl.when(j < na)
        def _():
            @pl.when(j + 1 < na)
            def _():
                atom_gather(j + 1, nxt_buf, nxt_sem)

            pltpu.make_async_copy(atom_hbm.at[pl.ds(0, ABLK)], buf, sem).wait()
            pltpu.sync_copy(buf, acc_a.at[aseg_all.at[j]], add=True)

    atom_gather(0, arows0, asem0)

    @pl.loop(0, ATOM_MAX, step=2)
    def _(j):
        atom_step(j, arows0, asem0, arows1, asem1)
        atom_step(j + 1, arows1, asem1, arows0, asem0)

    plsc.subcore_barrier()

    pltpu.sync_copy(acc_a.at[pl.ds(sid * 32, 32)], tmp_v)
    pltpu.sync_copy(tmp_v, out_a_hbm.at[cid, pl.ds(sid * 32, 32)])
    pltpu.sync_copy(acc_b.at[pl.ds(sid * 32, 32)], tmp_v)
    pltpu.sync_copy(tmp_v, out_b_hbm.at[cid, pl.ds(sid * 32, 32)])


def _sc_pool(bond_feats, b_ids, atom_feats, a_ids):
    mesh = plsc.VectorSubcoreMesh(core_axis_name="c", subcore_axis_name="s")
    f32 = jnp.float32
    i32 = jnp.int32
    cp = pltpu.CompilerParams()
    if "needs_layout_passes" in pltpu.CompilerParams.__dataclass_fields__:
        cp = dataclasses.replace(cp, needs_layout_passes=False)
    kern = pl.kernel(
        _sc_pool_body,
        compiler_params=cp,
        out_type=(
            jax.ShapeDtypeStruct((2, B, D), f32),
            jax.ShapeDtypeStruct((2, B, D), f32),
        ),
        mesh=mesh,
        scratch_types=[
            pltpu.VMEM_SHARED((B, D), f32),
            pltpu.VMEM_SHARED((B, D), f32),
            pltpu.VMEM((BBLK, D), f32),
            pltpu.VMEM((BBLK, D), f32),
            pltpu.VMEM((BBLK, D), f32),
            pltpu.VMEM((ABLK, D), f32),
            pltpu.VMEM((ABLK, D), f32),
            pltpu.VMEM((BBLK,), i32),
            pltpu.VMEM((BBLK,), i32),
            pltpu.VMEM((BBLK,), i32),
            pltpu.VMEM((BOND_MAX, BBLK), i32),
            pltpu.VMEM((ATOM_MAX, ABLK), i32),
            pltpu.VMEM((32, D), f32),
            pltpu.SemaphoreType.DMA,
            pltpu.SemaphoreType.DMA,
            pltpu.SemaphoreType.DMA,
            pltpu.SemaphoreType.DMA,
            pltpu.SemaphoreType.DMA,
            pltpu.SemaphoreType.DMA,
            pltpu.SemaphoreType.DMA,
            pltpu.SemaphoreType.DMA,
            pltpu.SemaphoreType.DMA,
        ],
    )
    return kern(bond_feats, b_ids, atom_feats, a_ids)


def _combine_body(pa_ref, pb_ref, g_ref, out_ref):
    out_ref[:, 0:D] = pa_ref[0] + pa_ref[1]
    out_ref[:, D : 2 * D] = pb_ref[0] + pb_ref[1]
    out_ref[:, 2 * D : 3 * D] = g_ref[...]


def _combine(pa, pb, g):
    return pl.pallas_call(
        _combine_body,
        out_shape=jax.ShapeDtypeStruct((B, 3 * D), jnp.float32),
    )(pa, pb, g)


def kernel(atom_feats, bond_feats, global_feats, atom_segment_ids, bond_segment_ids):
    a_ids = atom_segment_ids.astype(jnp.int32)
    b_ids = bond_segment_ids.astype(jnp.int32)
    pa, pb = _sc_pool(bond_feats, b_ids, atom_feats, a_ids)
    return _combine(pa, pb, global_feats)
